# Initial kernel scaffold; baseline (speedup 1.0000x reference)
import jax
import jax.numpy as jnp
from jax.experimental import pallas as pl


def kernel(lanes, road_id, lane_id, road_table, lane_table):
    raise NotImplementedError("write your pallas kernel here")


# speedup vs baseline: 1.0379x; 1.0379x over previous
import functools

import jax
import jax.numpy as jnp
from jax import lax
from jax.experimental import pallas as pl
from jax.experimental.pallas import tpu as pltpu
from jax.experimental.pallas import tpu_sc as plsc

_NC = 2
_NS = 16
_NW = _NC * _NS
_L = 16


@functools.lru_cache(maxsize=None)
def _build(n, feat, emb, dtype_name):
    dtype = jnp.dtype(dtype_name)
    out_w = feat + emb
    rows_per_w = n // _NW
    chunk = min(128, rows_per_w)
    nchunk = rows_per_w // chunk
    mesh = plsc.VectorSubcoreMesh(core_axis_name="c", subcore_axis_name="s")

    @functools.partial(
        pl.kernel,
        mesh=mesh,
        compiler_params=pltpu.CompilerParams(needs_layout_passes=False),
        out_type=jax.ShapeDtypeStruct((n * out_w,), dtype),
        scratch_types=[
            pltpu.VMEM((chunk,), jnp.int32),
            pltpu.VMEM((chunk,), jnp.int32),
            pltpu.VMEM((chunk * feat,), dtype),
            pltpu.VMEM((chunk, emb), dtype),
            pltpu.VMEM((chunk, emb), dtype),
            pltpu.VMEM((chunk * out_w,), dtype),
            pltpu.SemaphoreType.DMA,
            pltpu.SemaphoreType.DMA,
        ],
    )
    def sc_kernel(lanes_hbm, rid_hbm, lid_hbm, rtab_hbm, ltab_hbm, out_hbm,
                  rid_v, lid_v, feat_v, rrow_v, lrow_v, out_v, sem_r, sem_l):
        wid = lax.axis_index("s") * _NC + lax.axis_index("c")
        base = wid * rows_per_w
        for c in range(nchunk):
            row0 = base + c * chunk
            pltpu.sync_copy(rid_hbm.at[pl.ds(row0, chunk)], rid_v)
            pltpu.sync_copy(lid_hbm.at[pl.ds(row0, chunk)], lid_v)
            pltpu.sync_copy(lanes_hbm.at[pl.ds(row0 * feat, chunk * feat)],
                            feat_v)
            cp_r = pltpu.async_copy(rtab_hbm.at[rid_v], rrow_v, sem_r)
            cp_l = pltpu.async_copy(ltab_hbm.at[lid_v], lrow_v, sem_l)
            cp_r.wait()
            cp_l.wait()

            def add_row(rr, carry):
                b = rr * out_w + feat
                for j in range(emb // _L):
                    out_v[pl.ds(b + j * _L, _L)] = (
                        rrow_v[rr, pl.ds(j * _L, _L)]
                        + lrow_v[rr, pl.ds(j * _L, _L)])
                return carry

            lax.fori_loop(0, chunk, add_row, 0)

            iota = lax.iota(jnp.int32, _L)
            rows_per_vec = _L // feat
            tgt0 = iota
            for k in range(1, rows_per_vec):
                tgt0 = tgt0 + jnp.where(iota >= k * feat, emb, 0)
            step = _L + rows_per_vec * emb

            for i in range(chunk * feat // _L):
                vals = feat_v[pl.ds(i * _L, _L)]
                plsc.store_scatter(out_v, [tgt0 + i * step], vals)

            pltpu.sync_copy(out_v,
                            out_hbm.at[pl.ds(row0 * out_w, chunk * out_w)])

    return sc_kernel


def kernel(lanes, road_id, lane_id, road_table, lane_table):
    n, feat = lanes.shape
    emb = road_table.shape[1]
    fn = _build(n, feat, emb, str(road_table.dtype))
    out = fn(lanes.reshape(-1),
             road_id.astype(jnp.int32),
             lane_id.astype(jnp.int32),
             road_table,
             lane_table)
    return out.reshape(n, feat + emb)


# speedup vs baseline: 1.1489x; 1.1070x over previous
import functools

import jax
import jax.numpy as jnp
from jax import lax
from jax.experimental import pallas as pl
from jax.experimental.pallas import tpu as pltpu
from jax.experimental.pallas import tpu_sc as plsc

_NC = 2
_NS = 16
_NW = _NC * _NS
_L = 16


@functools.lru_cache(maxsize=None)
def _build(n, feat, emb, dtype_name):
    dtype = jnp.dtype(dtype_name)
    out_w = feat + emb
    rows_per_w = n // _NW
    chunk = min(128, rows_per_w)
    nchunk = rows_per_w // chunk
    nbuf = 2
    mesh = plsc.VectorSubcoreMesh(core_axis_name="c", subcore_axis_name="s")

    @functools.partial(
        pl.kernel,
        mesh=mesh,
        compiler_params=pltpu.CompilerParams(
            needs_layout_passes=False, use_tc_tiling_on_sc=False),
        out_type=jax.ShapeDtypeStruct((n, out_w), dtype),
        scratch_types=[
            *[pltpu.VMEM((chunk,), jnp.int32) for _ in range(nbuf)],
            *[pltpu.VMEM((chunk,), jnp.int32) for _ in range(nbuf)],
            *[pltpu.VMEM((chunk, emb), dtype) for _ in range(nbuf)],
            *[pltpu.VMEM((chunk, emb), dtype) for _ in range(nbuf)],
            *[pltpu.VMEM((chunk, out_w), dtype) for _ in range(nbuf)],
            *[pltpu.SemaphoreType.DMA for _ in range(nbuf)],
            *[pltpu.SemaphoreType.DMA for _ in range(nbuf)],
            *[pltpu.SemaphoreType.DMA for _ in range(nbuf)],
            pltpu.SemaphoreType.DMA,
        ],
    )
    def sc_kernel(lanes_hbm, rid_hbm, lid_hbm, rtab_hbm, ltab_hbm, out_hbm,
                  *scr):
        rid_v = scr[0:nbuf]
        lid_v = scr[nbuf:2 * nbuf]
        rrow_v = scr[2 * nbuf:3 * nbuf]
        lrow_v = scr[3 * nbuf:4 * nbuf]
        out_v = scr[4 * nbuf:5 * nbuf]
        sem_r = scr[5 * nbuf:6 * nbuf]
        sem_l = scr[6 * nbuf:7 * nbuf]
        sem_o = scr[7 * nbuf:8 * nbuf]
        sem_f = scr[8 * nbuf]

        wid = lax.axis_index("s") * _NC + lax.axis_index("c")
        base = wid * rows_per_w

        def start_gathers(c):
            b = c % nbuf
            row0 = base + c * chunk
            pltpu.sync_copy(rid_hbm.at[pl.ds(row0, chunk)], rid_v[b])
            pltpu.sync_copy(lid_hbm.at[pl.ds(row0, chunk)], lid_v[b])
            cp_r = pltpu.async_copy(rtab_hbm.at[rid_v[b]], rrow_v[b], sem_r[b])
            cp_l = pltpu.async_copy(ltab_hbm.at[lid_v[b]], lrow_v[b], sem_l[b])
            return cp_r, cp_l

        gathers = {0: start_gathers(0)}
        out_cps = {}
        for c in range(nchunk):
            b = c % nbuf
            row0 = base + c * chunk
            if c + 1 < nchunk:
                gathers[c + 1] = start_gathers(c + 1)
            if c - (nbuf - 1) >= 0:
                out_cps.pop(c - (nbuf - 1)).wait()
            cp_f = pltpu.async_copy(lanes_hbm.at[pl.ds(row0, chunk)],
                                    out_v[b].at[:, pl.ds(0, feat)], sem_f)
            cp_r, cp_l = gathers.pop(c)
            cp_r.wait()
            cp_l.wait()

            def add_row(rr, carry, _b=b):
                for j in range(emb // _L):
                    out_v[_b][rr, pl.ds(feat + j * _L, _L)] = (
                        rrow_v[_b][rr, pl.ds(j * _L, _L)]
                        + lrow_v[_b][rr, pl.ds(j * _L, _L)])
                return carry

            lax.fori_loop(0, chunk, add_row, 0)
            cp_f.wait()
            out_cps[c] = pltpu.async_copy(
                out_v[b], out_hbm.at[pl.ds(row0, chunk)], sem_o[b])
        for c in sorted(out_cps):
            out_cps.pop(c).wait()

    return sc_kernel


def kernel(lanes, road_id, lane_id, road_table, lane_table):
    n, feat = lanes.shape
    emb = road_table.shape[1]
    fn = _build(n, feat, emb, str(road_table.dtype))
    return fn(lanes,
              road_id.astype(jnp.int32),
              lane_id.astype(jnp.int32),
              road_table,
              lane_table)


# speedup vs baseline: 1.9832x; 1.7261x over previous
import functools

import jax
import jax.numpy as jnp
from jax import lax
from jax.experimental import pallas as pl
from jax.experimental.pallas import tpu as pltpu
from jax.experimental.pallas import tpu_sc as plsc

_NC = 2
_NS = 16
_NW = _NC * _NS
_L = 16


@functools.lru_cache(maxsize=None)
def _build(n, feat, emb, dtype_name):
    dtype = jnp.dtype(dtype_name)
    out_w = feat + emb
    rows_per_w = n // _NW
    chunk = min(64, rows_per_w)
    nchunk = rows_per_w // chunk
    nbuf = 2
    mesh = plsc.VectorSubcoreMesh(core_axis_name="c", subcore_axis_name="s")

    n_slice = emb // _L - 1
    tail_src = n_slice * _L

    @functools.partial(
        pl.kernel,
        mesh=mesh,
        compiler_params=pltpu.CompilerParams(needs_layout_passes=False),
        out_type=jax.ShapeDtypeStruct((n, out_w), dtype),
        scratch_types=[
            *[pltpu.VMEM((chunk,), jnp.int32) for _ in range(nbuf)],
            *[pltpu.VMEM((chunk,), jnp.int32) for _ in range(nbuf)],
            *[pltpu.VMEM((chunk, feat), dtype) for _ in range(nbuf)],
            *[pltpu.VMEM((chunk, emb), dtype) for _ in range(nbuf)],
            *[pltpu.VMEM((chunk, emb), dtype) for _ in range(nbuf)],
            *[pltpu.VMEM((chunk, out_w), dtype) for _ in range(nbuf)],
            *[pltpu.SemaphoreType.DMA for _ in range(nbuf)],
            *[pltpu.SemaphoreType.DMA for _ in range(nbuf)],
            *[pltpu.SemaphoreType.DMA for _ in range(nbuf)],
            *[pltpu.SemaphoreType.DMA for _ in range(nbuf)],
        ],
    )
    def sc_kernel(lanes_hbm, rid_hbm, lid_hbm, rtab_hbm, ltab_hbm, out_hbm,
                  *scr):
        rid_v = scr[0:nbuf]
        lid_v = scr[nbuf:2 * nbuf]
        feat_v = scr[2 * nbuf:3 * nbuf]
        rrow_v = scr[3 * nbuf:4 * nbuf]
        lrow_v = scr[4 * nbuf:5 * nbuf]
        out_v = scr[5 * nbuf:6 * nbuf]
        sem_r = scr[6 * nbuf:7 * nbuf]
        sem_l = scr[7 * nbuf:8 * nbuf]
        sem_o = scr[8 * nbuf:9 * nbuf]
        sem_f = scr[9 * nbuf:10 * nbuf]

        wid = lax.axis_index("s") * _NC + lax.axis_index("c")
        base = wid * rows_per_w

        iota = lax.iota(jnp.int32, _L)
        rvec0 = jnp.where(iota >= feat, 1, 0)
        for k in range(2, _L // feat):
            rvec0 = rvec0 + jnp.where(iota >= k * feat, 1, 0)
        cvec_f = iota - rvec0 * feat
        rstep = _L // feat
        cvec_t = iota + (out_w - _L)

        def start_fetch(c):
            b = c % nbuf
            row0 = base + c * chunk
            pltpu.sync_copy(rid_hbm.at[pl.ds(row0, chunk)], rid_v[b])
            pltpu.sync_copy(lid_hbm.at[pl.ds(row0, chunk)], lid_v[b])
            cp_r = pltpu.async_copy(rtab_hbm.at[rid_v[b]], rrow_v[b], sem_r[b])
            cp_l = pltpu.async_copy(ltab_hbm.at[lid_v[b]], lrow_v[b], sem_l[b])
            cp_f = pltpu.async_copy(lanes_hbm.at[pl.ds(row0, chunk)],
                                    feat_v[b], sem_f[b])
            return cp_r, cp_l, cp_f

        fetches = {0: start_fetch(0)}
        out_cps = {}
        for c in range(nchunk):
            b = c % nbuf
            row0 = base + c * chunk
            if c + 1 < nchunk:
                fetches[c + 1] = start_fetch(c + 1)
            if c - nbuf >= 0:
                out_cps.pop(c - nbuf).wait()
            cp_r, cp_l, cp_f = fetches.pop(c)
            cp_f.wait()
            for i in range(chunk * feat // _L):
                rv = rvec0 + i * rstep
                vals = plsc.load_gather(feat_v[b], [rv, cvec_f])
                plsc.store_scatter(out_v[b], [rv, cvec_f], vals)
            cp_r.wait()
            cp_l.wait()

            @plsc.parallel_loop(0, chunk, carry=iota * 0)
            def add_row(rr, rv, _b=b):
                for j in range(n_slice):
                    out_v[_b][rr, pl.ds(feat + j * _L, _L)] = (
                        rrow_v[_b][rr, pl.ds(j * _L, _L)]
                        + lrow_v[_b][rr, pl.ds(j * _L, _L)])
                tail = (rrow_v[_b][rr, pl.ds(tail_src, _L)]
                        + lrow_v[_b][rr, pl.ds(tail_src, _L)])
                plsc.store_scatter(out_v[_b], [rv, cvec_t], tail)
                return rv + 1

            out_cps[c] = pltpu.async_copy(
                out_v[b], out_hbm.at[pl.ds(row0, chunk)], sem_o[b])
        for c in sorted(out_cps):
            out_cps.pop(c).wait()

    return sc_kernel


def kernel(lanes, road_id, lane_id, road_table, lane_table):
    n, feat = lanes.shape
    emb = road_table.shape[1]
    fn = _build(n, feat, emb, str(road_table.dtype))
    return fn(lanes,
              road_id.astype(jnp.int32),
              lane_id.astype(jnp.int32),
              road_table,
              lane_table)
